# baseline (speedup 1.0000x reference)
import jax
import jax.numpy as jnp
from jax import lax
from jax.experimental import pallas as pl
from jax.experimental.pallas import tpu as pltpu
from jax.experimental.pallas import tpu_sc as plsc

_VOCAB = 1000000
_D = 32
_B = 16384
_L = 20

_NW = 32
_BW = _B // _NW
_CH = 128
_NCH = _BW // _CH
_PPC = _CH * _L
_ROWS = _PPC // 128
_GRP = _PPC // 16


def _sc_scores(tw_hbm, cp_hbm, inp_hbm, node_hbm, out_hbm,
               tw_v, cp_v, inp_v, node_v, scores_v, sem):
    c = lax.axis_index("c")
    s = lax.axis_index("s")
    wid = s * 2 + c

    for ci in range(_NCH):
        b0 = wid * _BW + ci * _CH
        p0 = b0 * _L
        pltpu.sync_copy(tw_hbm.at[pl.ds(b0, _CH)], tw_v)
        pltpu.sync_copy(cp_hbm.at[pl.ds(p0, _PPC)], cp_v)
        descs = [pltpu.async_copy(inp_hbm.at[tw_v], inp_v, sem)]
        for j in range(_ROWS):
            descs.append(pltpu.async_copy(node_hbm.at[cp_v.at[pl.ds(j * 128, 128)]],
                                          node_v.at[pl.ds(j * 128, 128)],
                                          sem))
        for dd in descs:
            dd.wait()

        lvec = jnp.full((16,), _L, jnp.int32)

        def group(g, carry):
            p0 = g * 16
            row_ids = p0 + lax.iota(jnp.int32, 16)
            b_ids = lax.div(row_ids, lvec)
            acc = jnp.zeros((16,), jnp.float32)
            for d in range(_D):
                dsp = jnp.full((16,), d, jnp.int32)
                nv = plsc.load_gather(node_v, [row_ids, dsp])
                iv = plsc.load_gather(inp_v, [b_ids, dsp])
                acc = acc + nv * iv
            scores_v[pl.ds(p0, 16)] = acc
            return carry

        lax.fori_loop(0, _GRP, group, 0)
        pltpu.sync_copy(scores_v, out_hbm.at[pl.ds(b0 * _L, _PPC)])


_sc_call = pl.kernel(
    _sc_scores,
    mesh=plsc.VectorSubcoreMesh(core_axis_name="c", subcore_axis_name="s"),
    out_type=jax.ShapeDtypeStruct((_B * _L,), jnp.float32),
    scratch_types=[
        pltpu.VMEM((_CH,), jnp.int32),
        pltpu.VMEM((_PPC,), jnp.int32),
        pltpu.VMEM((_CH, _D), jnp.float32),
        pltpu.VMEM((_PPC, _D), jnp.float32),
        pltpu.VMEM((_PPC,), jnp.float32),
        pltpu.SemaphoreType.DMA,
    ],
    compiler_params=pltpu.CompilerParams(
        needs_layout_passes=False, use_tc_tiling_on_sc=False),
)


def _tc_loss(scores_ref, codes_ref, out_ref):
    sc = scores_ref[...]
    cd = codes_ref[...]
    cf = cd.astype(jnp.float32)
    p = 1.0 / (1.0 + jnp.exp(-sc))
    t = cf * jnp.log(p + 1e-7) + (1.0 - cf) * jnp.log(1.0 - p + 1e-7)
    total = jnp.sum(t)
    msum = jnp.sum(jnp.where(cd != -1, 1.0, 0.0))
    out_ref[0, 0] = -total * (msum / float(_B * _L))


def kernel(target_words, context_codes, context_points, in_embed, node_embed):
    tw = target_words.astype(jnp.int32)
    cp = context_points.astype(jnp.int32).reshape(_B * _L)
    scores = _sc_call(tw, cp, in_embed, node_embed)
    scores2 = scores.reshape(_B * _L // 128, 128)
    codes2 = context_codes.astype(jnp.int32).reshape(_B * _L // 128, 128)
    out = pl.pallas_call(
        _tc_loss,
        out_shape=jax.ShapeDtypeStruct((1, 1), jnp.float32),
        out_specs=pl.BlockSpec(memory_space=pltpu.SMEM),
    )(scores2, codes2)
    return out[0, 0]


# speedup vs baseline: 2.1088x; 2.1088x over previous
import jax
import jax.numpy as jnp
from jax import lax
from jax.experimental import pallas as pl
from jax.experimental.pallas import tpu as pltpu
from jax.experimental.pallas import tpu_sc as plsc

_VOCAB = 1000000
_D = 32
_B = 16384
_L = 20

_WI = 1 << 18
_WN = 1 << 19
_J = 2048

_NW = 32
_BW = _B // _NW
_CH = 128
_NCH = _BW // _CH
_PPC = _CH * _L
_ROWS = _PPC // 128
_GRP = _PPC // 16


def _tr_body(a_ref, b_ref, c_ref, d_ref, out_ref):
    x = jnp.concatenate(
        [a_ref[...], b_ref[...], c_ref[...], d_ref[...]], axis=0)
    out_ref[...] = x.T


def _make_tr(w, ncols):
    m = w // _J
    last = (ncols + _J - 1) // _J - 1
    return pl.pallas_call(
        _tr_body,
        grid=(m,),
        in_specs=[
            pl.BlockSpec((32, _J), lambda i: (0, i)),
            pl.BlockSpec((32, _J), lambda i: (0, i + m)),
            pl.BlockSpec((32, _J), lambda i: (0, i + 2 * m)),
            pl.BlockSpec((32, _J), lambda i: (0, jnp.minimum(i + 3 * m, last))),
        ],
        out_specs=pl.BlockSpec((_J, 128), lambda i: (i, 0)),
        out_shape=jax.ShapeDtypeStruct((w, 128), jnp.float32),
    )


_tr_in = _make_tr(_WI, _VOCAB)
_tr_node = _make_tr(_WN, 2 * _VOCAB - 1)


def _sc_scores(tw_hbm, cp_hbm, inp_hbm, node_hbm, out_hbm,
               tw_v, cp_v, inp_v, node_v, scores_v, sem):
    c = lax.axis_index("c")
    s = lax.axis_index("s")
    wid = s * 2 + c

    for ci in range(_NCH):
        b0 = wid * _BW + ci * _CH
        p0 = b0 * _L
        pltpu.sync_copy(tw_hbm.at[pl.ds(b0, _CH)], tw_v)
        pltpu.sync_copy(cp_hbm.at[pl.ds(p0, _PPC)], cp_v)
        descs = [pltpu.async_copy(inp_hbm.at[tw_v], inp_v, sem)]
        for j in range(_ROWS):
            descs.append(pltpu.async_copy(node_hbm.at[cp_v.at[pl.ds(j * 128, 128)]],
                                          node_v.at[pl.ds(j * 128, 128)],
                                          sem))
        for dd in descs:
            dd.wait()

        lvec = jnp.full((16,), _L, jnp.int32)

        def group(g, carry):
            q0 = g * 16
            row_ids = q0 + lax.iota(jnp.int32, 16)
            b_ids = lax.div(row_ids, lvec)
            acc = jnp.zeros((16,), jnp.float32)
            for d in range(_D):
                dsp = jnp.full((16,), d, jnp.int32)
                nv = plsc.load_gather(node_v, [row_ids, dsp])
                iv = plsc.load_gather(inp_v, [b_ids, dsp])
                acc = acc + nv * iv
            scores_v[pl.ds(q0, 16)] = acc
            return carry

        lax.fori_loop(0, _GRP, group, 0)
        pltpu.sync_copy(scores_v, out_hbm.at[pl.ds(b0 * _L, _PPC)])


_sc_call = pl.kernel(
    _sc_scores,
    mesh=plsc.VectorSubcoreMesh(core_axis_name="c", subcore_axis_name="s"),
    out_type=jax.ShapeDtypeStruct((_B * _L,), jnp.float32),
    scratch_types=[
        pltpu.VMEM((_CH,), jnp.int32),
        pltpu.VMEM((_PPC,), jnp.int32),
        pltpu.VMEM((_CH, _D), jnp.float32),
        pltpu.VMEM((_PPC, _D), jnp.float32),
        pltpu.VMEM((_PPC,), jnp.float32),
        pltpu.SemaphoreType.DMA,
    ],
    compiler_params=pltpu.CompilerParams(
        needs_layout_passes=False, use_tc_tiling_on_sc=False),
)


def _tc_loss(scores_ref, codes_ref, out_ref):
    sc = scores_ref[...]
    cd = codes_ref[...]
    cf = cd.astype(jnp.float32)
    p = 1.0 / (1.0 + jnp.exp(-sc))
    t = cf * jnp.log(p + 1e-7) + (1.0 - cf) * jnp.log(1.0 - p + 1e-7)
    total = jnp.sum(t)
    msum = jnp.sum(jnp.where(cd != -1, 1.0, 0.0))
    out_ref[0, 0] = -total * (msum / float(_B * _L))


def kernel(target_words, context_codes, context_points, in_embed, node_embed):
    tw = target_words.astype(jnp.int32)
    cp = context_points.astype(jnp.int32).reshape(_B * _L)
    tw_p = ((tw & (_WI - 1)) << 2) | (tw >> 18)
    cp_p = ((cp & (_WN - 1)) << 2) | (cp >> 19)
    inT = in_embed.T
    nodeT = node_embed.T
    in_lin = _tr_in(inT, inT, inT, inT).reshape(4 * _WI, _D)
    node_lin = _tr_node(nodeT, nodeT, nodeT, nodeT).reshape(4 * _WN, _D)
    scores = _sc_call(tw_p, cp_p, in_lin, node_lin)
    scores2 = scores.reshape(_B * _L // 128, 128)
    codes2 = context_codes.astype(jnp.int32).reshape(_B * _L // 128, 128)
    out = pl.pallas_call(
        _tc_loss,
        out_shape=jax.ShapeDtypeStruct((1, 1), jnp.float32),
        out_specs=pl.BlockSpec(memory_space=pltpu.SMEM),
    )(scores2, codes2)
    return out[0, 0]


# speedup vs baseline: 2.1189x; 1.0048x over previous
import jax
import jax.numpy as jnp
from jax import lax
from jax.experimental import pallas as pl
from jax.experimental.pallas import tpu as pltpu
from jax.experimental.pallas import tpu_sc as plsc

_VOCAB = 1000000
_D = 32
_B = 16384
_L = 20

_WI = 1 << 18
_WN = 1 << 19
_J = 2048

_NW = 32
_BW = _B // _NW
_CH = 64
_NCH = _BW // _CH
_PPC = _CH * _L
_ROWS = _PPC // 128
_GRP = _PPC // 16


def _tr_body(a_ref, b_ref, c_ref, d_ref, out_ref):
    x = jnp.concatenate(
        [a_ref[...], b_ref[...], c_ref[...], d_ref[...]], axis=0)
    out_ref[...] = x.T


def _make_tr(w, ncols):
    m = w // _J
    last = (ncols + _J - 1) // _J - 1
    return pl.pallas_call(
        _tr_body,
        grid=(m,),
        in_specs=[
            pl.BlockSpec((32, _J), lambda i: (0, i)),
            pl.BlockSpec((32, _J), lambda i: (0, i + m)),
            pl.BlockSpec((32, _J), lambda i: (0, i + 2 * m)),
            pl.BlockSpec((32, _J), lambda i: (0, jnp.minimum(i + 3 * m, last))),
        ],
        out_specs=pl.BlockSpec((_J, 128), lambda i: (i, 0)),
        out_shape=jax.ShapeDtypeStruct((w, 128), jnp.float32),
    )


_tr_in = _make_tr(_WI, _VOCAB)
_tr_node = _make_tr(_WN, 2 * _VOCAB - 1)


def _sc_scores(tw_hbm, cp_hbm, inp_hbm, node_hbm, out_hbm,
               tw_v0, tw_v1, cp_v0, cp_v1, inp_v0, inp_v1,
               node_v0, node_v1, sc_v0, sc_v1, sem0, sem1):
    c = lax.axis_index("c")
    s = lax.axis_index("s")
    wid = s * 2 + c
    tw_v = (tw_v0, tw_v1)
    cp_v = (cp_v0, cp_v1)
    inp_v = (inp_v0, inp_v1)
    node_v = (node_v0, node_v1)
    sc_v = (sc_v0, sc_v1)
    sems = (sem0, sem1)

    def start(ci):
        k = ci % 2
        b0 = wid * _BW + ci * _CH
        pltpu.sync_copy(tw_hbm.at[pl.ds(b0, _CH)], tw_v[k])
        pltpu.sync_copy(cp_hbm.at[pl.ds(b0 * _L, _PPC)], cp_v[k])
        ds = [pltpu.async_copy(inp_hbm.at[tw_v[k]], inp_v[k], sems[k])]
        for j in range(_ROWS):
            ds.append(pltpu.async_copy(
                node_hbm.at[cp_v[k].at[pl.ds(j * 128, 128)]],
                node_v[k].at[pl.ds(j * 128, 128)], sems[k]))
        return ds

    lvec = jnp.full((16,), _L, jnp.int32)
    cur = start(0)
    for ci in range(_NCH):
        k = ci % 2
        nxt = start(ci + 1) if ci + 1 < _NCH else None
        for dd in cur:
            dd.wait()

        def group(g, carry, k=k):
            q0 = g * 16
            row_ids = q0 + lax.iota(jnp.int32, 16)
            b_ids = lax.div(row_ids, lvec)
            acc = jnp.zeros((16,), jnp.float32)
            for d in range(_D):
                dsp = jnp.full((16,), d, jnp.int32)
                nv = plsc.load_gather(node_v[k], [row_ids, dsp])
                iv = plsc.load_gather(inp_v[k], [b_ids, dsp])
                acc = acc + nv * iv
            sc_v[k][pl.ds(q0, 16)] = acc
            return carry

        lax.fori_loop(0, _GRP, group, 0)
        b0 = wid * _BW + ci * _CH
        pltpu.sync_copy(sc_v[k], out_hbm.at[pl.ds(b0 * _L, _PPC)])
        cur = nxt


_sc_call = pl.kernel(
    _sc_scores,
    mesh=plsc.VectorSubcoreMesh(core_axis_name="c", subcore_axis_name="s"),
    out_type=jax.ShapeDtypeStruct((_B * _L,), jnp.float32),
    scratch_types=[
        pltpu.VMEM((_CH,), jnp.int32),
        pltpu.VMEM((_CH,), jnp.int32),
        pltpu.VMEM((_PPC,), jnp.int32),
        pltpu.VMEM((_PPC,), jnp.int32),
        pltpu.VMEM((_CH, _D), jnp.float32),
        pltpu.VMEM((_CH, _D), jnp.float32),
        pltpu.VMEM((_PPC, _D), jnp.float32),
        pltpu.VMEM((_PPC, _D), jnp.float32),
        pltpu.VMEM((_PPC,), jnp.float32),
        pltpu.VMEM((_PPC,), jnp.float32),
        pltpu.SemaphoreType.DMA,
        pltpu.SemaphoreType.DMA,
    ],
    compiler_params=pltpu.CompilerParams(
        needs_layout_passes=False, use_tc_tiling_on_sc=False),
)


def _tc_loss(scores_ref, codes_ref, out_ref):
    sc = scores_ref[...]
    cd = codes_ref[...]
    cf = cd.astype(jnp.float32)
    p = 1.0 / (1.0 + jnp.exp(-sc))
    t = cf * jnp.log(p + 1e-7) + (1.0 - cf) * jnp.log(1.0 - p + 1e-7)
    total = jnp.sum(t)
    msum = jnp.sum(jnp.where(cd != -1, 1.0, 0.0))
    out_ref[0, 0] = -total * (msum / float(_B * _L))


def kernel(target_words, context_codes, context_points, in_embed, node_embed):
    tw = target_words.astype(jnp.int32)
    cp = context_points.astype(jnp.int32).reshape(_B * _L)
    tw_p = ((tw & (_WI - 1)) << 2) | (tw >> 18)
    cp_p = ((cp & (_WN - 1)) << 2) | (cp >> 19)
    inT = in_embed.T
    nodeT = node_embed.T
    in_lin = _tr_in(inT, inT, inT, inT).reshape(4 * _WI, _D)
    node_lin = _tr_node(nodeT, nodeT, nodeT, nodeT).reshape(4 * _WN, _D)
    scores = _sc_call(tw_p, cp_p, in_lin, node_lin)
    scores2 = scores.reshape(_B * _L // 128, 128)
    codes2 = context_codes.astype(jnp.int32).reshape(_B * _L // 128, 128)
    out = pl.pallas_call(
        _tc_loss,
        out_shape=jax.ShapeDtypeStruct((1, 1), jnp.float32),
        out_specs=pl.BlockSpec(memory_space=pltpu.SMEM),
    )(scores2, codes2)
    return out[0, 0]


# speedup vs baseline: 2.1487x; 1.0141x over previous
import jax
import jax.numpy as jnp
from jax import lax
from jax.experimental import pallas as pl
from jax.experimental.pallas import tpu as pltpu
from jax.experimental.pallas import tpu_sc as plsc

_VOCAB = 1000000
_D = 32
_B = 16384
_L = 20

_WI = 1 << 18
_WN = 1 << 19
_J = 2048

_NW = 32
_BW = _B // _NW
_CH = 64
_NCH = _BW // _CH
_PPC = _CH * _L
_ROWS = _PPC // 128
_GRP = _PPC // 16


def _tr_body(a_ref, b_ref, c_ref, d_ref, out_ref):
    x = jnp.concatenate(
        [a_ref[...], b_ref[...], c_ref[...], d_ref[...]], axis=0)
    out_ref[...] = x.T


def _make_tr(w, ncols):
    m = w // _J
    last = (ncols + _J - 1) // _J - 1
    return pl.pallas_call(
        _tr_body,
        grid=(m,),
        in_specs=[
            pl.BlockSpec((32, _J), lambda i: (0, i)),
            pl.BlockSpec((32, _J), lambda i: (0, i + m)),
            pl.BlockSpec((32, _J), lambda i: (0, i + 2 * m)),
            pl.BlockSpec((32, _J), lambda i: (0, jnp.minimum(i + 3 * m, last))),
        ],
        out_specs=pl.BlockSpec((_J, 128), lambda i: (i, 0)),
        out_shape=jax.ShapeDtypeStruct((w, 128), jnp.float32),
    )


_tr_in = _make_tr(_WI, _VOCAB)
_tr_node = _make_tr(_WN, 2 * _VOCAB - 1)


def _sc_scores(tw_hbm, cp_hbm, inp_hbm, node_hbm, out_hbm,
               tw_v0, tw_v1, cp_v0, cp_v1, inp_v0, inp_v1,
               node_v0, node_v1, sc_v0, sc_v1, sem0, sem1):
    c = lax.axis_index("c")
    s = lax.axis_index("s")
    wid = s * 2 + c
    tw_v = (tw_v0, tw_v1)
    cp_v = (cp_v0, cp_v1)
    inp_v = (inp_v0, inp_v1)
    node_v = (node_v0, node_v1)
    sc_v = (sc_v0, sc_v1)
    sems = (sem0, sem1)

    def start(ci):
        k = ci % 2
        b0 = wid * _BW + ci * _CH
        pltpu.sync_copy(tw_hbm.at[pl.ds(b0, _CH)], tw_v[k])
        pltpu.sync_copy(cp_hbm.at[pl.ds(b0 * _L, _PPC)], cp_v[k])
        ds = [pltpu.async_copy(inp_hbm.at[tw_v[k]], inp_v[k], sems[k])]
        for j in range(_ROWS):
            ds.append(pltpu.async_copy(
                node_hbm.at[cp_v[k].at[pl.ds(j * 128, 128)]],
                node_v[k].at[pl.ds(j * 128, 128)], sems[k]))
        return ds

    lvec = jnp.full((16,), _L, jnp.int32)
    cur = start(0)
    for ci in range(_NCH):
        k = ci % 2
        nxt = start(ci + 1) if ci + 1 < _NCH else None
        for dd in cur:
            dd.wait()

        @plsc.parallel_loop(0, _GRP, unroll=2)
        def group(g, k=k):
            q0 = g * 16
            row_ids = q0 + lax.iota(jnp.int32, 16)
            b_ids = lax.div(row_ids, lvec)
            accs = [jnp.zeros((16,), jnp.float32) for _ in range(4)]
            for d in range(_D):
                dsp = jnp.full((16,), d, jnp.int32)
                nv = plsc.load_gather(node_v[k], [row_ids, dsp])
                iv = plsc.load_gather(inp_v[k], [b_ids, dsp])
                accs[d % 4] = accs[d % 4] + nv * iv
            sc_v[k][pl.ds(q0, 16)] = (accs[0] + accs[1]) + (accs[2] + accs[3])
        b0 = wid * _BW + ci * _CH
        pltpu.sync_copy(sc_v[k], out_hbm.at[pl.ds(b0 * _L, _PPC)])
        cur = nxt


_sc_call = pl.kernel(
    _sc_scores,
    mesh=plsc.VectorSubcoreMesh(core_axis_name="c", subcore_axis_name="s"),
    out_type=jax.ShapeDtypeStruct((_B * _L,), jnp.float32),
    scratch_types=[
        pltpu.VMEM((_CH,), jnp.int32),
        pltpu.VMEM((_CH,), jnp.int32),
        pltpu.VMEM((_PPC,), jnp.int32),
        pltpu.VMEM((_PPC,), jnp.int32),
        pltpu.VMEM((_CH, _D), jnp.float32),
        pltpu.VMEM((_CH, _D), jnp.float32),
        pltpu.VMEM((_PPC, _D), jnp.float32),
        pltpu.VMEM((_PPC, _D), jnp.float32),
        pltpu.VMEM((_PPC,), jnp.float32),
        pltpu.VMEM((_PPC,), jnp.float32),
        pltpu.SemaphoreType.DMA,
        pltpu.SemaphoreType.DMA,
    ],
    compiler_params=pltpu.CompilerParams(
        needs_layout_passes=False, use_tc_tiling_on_sc=False),
)


def _tc_loss(scores_ref, codes_ref, out_ref):
    sc = scores_ref[...]
    cd = codes_ref[...]
    cf = cd.astype(jnp.float32)
    p = 1.0 / (1.0 + jnp.exp(-sc))
    t = cf * jnp.log(p + 1e-7) + (1.0 - cf) * jnp.log(1.0 - p + 1e-7)
    total = jnp.sum(t)
    msum = jnp.sum(jnp.where(cd != -1, 1.0, 0.0))
    out_ref[0, 0] = -total * (msum / float(_B * _L))


def kernel(target_words, context_codes, context_points, in_embed, node_embed):
    tw = target_words.astype(jnp.int32)
    cp = context_points.astype(jnp.int32).reshape(_B * _L)
    tw_p = ((tw & (_WI - 1)) << 2) | (tw >> 18)
    cp_p = ((cp & (_WN - 1)) << 2) | (cp >> 19)
    inT = in_embed.T
    nodeT = node_embed.T
    in_lin = _tr_in(inT, inT, inT, inT).reshape(4 * _WI, _D)
    node_lin = _tr_node(nodeT, nodeT, nodeT, nodeT).reshape(4 * _WN, _D)
    scores = _sc_call(tw_p, cp_p, in_lin, node_lin)
    scores2 = scores.reshape(_B * _L // 128, 128)
    codes2 = context_codes.astype(jnp.int32).reshape(_B * _L // 128, 128)
    out = pl.pallas_call(
        _tc_loss,
        out_shape=jax.ShapeDtypeStruct((1, 1), jnp.float32),
        out_specs=pl.BlockSpec(memory_space=pltpu.SMEM),
    )(scores2, codes2)
    return out[0, 0]


# speedup vs baseline: 2.8389x; 1.3212x over previous
import jax
import jax.numpy as jnp
from jax import lax
from jax.experimental import pallas as pl
from jax.experimental.pallas import tpu as pltpu
from jax.experimental.pallas import tpu_sc as plsc

_VOCAB = 1000000
_D = 32
_B = 16384
_L = 20

_WI = 1 << 18
_WN = 1 << 19
_J = 2048

_NW = 32
_BW = _B // _NW
_CH = 64
_NCH = _BW // _CH
_PPC = _CH * _L
_ROWS = _PPC // 128
_GRP = _PPC // 16


def _tr_body(a_ref, b_ref, c_ref, d_ref, out_ref):
    x = jnp.concatenate(
        [a_ref[...], b_ref[...], c_ref[...], d_ref[...]], axis=0)
    out_ref[...] = x.T


def _make_tr(w, ncols):
    m = w // _J
    last = (ncols + _J - 1) // _J - 1
    return pl.pallas_call(
        _tr_body,
        grid=(m,),
        in_specs=[
            pl.BlockSpec((32, _J), lambda i: (0, i)),
            pl.BlockSpec((32, _J), lambda i: (0, i + m)),
            pl.BlockSpec((32, _J), lambda i: (0, i + 2 * m)),
            pl.BlockSpec((32, _J), lambda i: (0, jnp.minimum(i + 3 * m, last))),
        ],
        out_specs=pl.BlockSpec((_J, 128), lambda i: (i, 0)),
        out_shape=jax.ShapeDtypeStruct((w, 128), jnp.float32),
    )


_tr_in = _make_tr(_WI, _VOCAB)
_tr_node = _make_tr(_WN, 2 * _VOCAB - 1)


def _sc_scores(tw_hbm, cp_hbm, inp_hbm, node_hbm, out_hbm,
               tw_v0, tw_v1, cp_v0, cp_v1, inp_v0, inp_v1,
               node_v0, node_v1, sc_v0, sc_v1, sem0, sem1):
    c = lax.axis_index("c")
    s = lax.axis_index("s")
    wid = s * 2 + c
    tw_v = (tw_v0, tw_v1)
    cp_v = (cp_v0, cp_v1)
    inp_v = (inp_v0, inp_v1)
    node_v = (node_v0, node_v1)
    sc_v = (sc_v0, sc_v1)
    sems = (sem0, sem1)

    def start(ci):
        k = ci % 2
        b0 = wid * _BW + ci * _CH
        pltpu.sync_copy(tw_hbm.at[pl.ds(b0, _CH)], tw_v[k])
        pltpu.sync_copy(cp_hbm.at[pl.ds(b0 * _L, _PPC)], cp_v[k])
        ds = [pltpu.async_copy(inp_hbm.at[tw_v[k]], inp_v[k], sems[k])]
        for j in range(_ROWS):
            ds.append(pltpu.async_copy(
                node_hbm.at[cp_v[k].at[pl.ds(j * 128, 128)]],
                node_v[k].at[pl.ds(j * 128, 128)], sems[k]))
        return ds

    lvec = jnp.full((16,), _L, jnp.int32)
    cur = start(0)
    for ci in range(_NCH):
        k = ci % 2
        nxt = start(ci + 1) if ci + 1 < _NCH else None
        for dd in cur:
            dd.wait()

        @plsc.parallel_loop(0, _GRP, unroll=2)
        def group(g, k=k):
            q0 = g * 16
            ii = lax.iota(jnp.int32, 16)
            row_ids = q0 + ii
            b_ids = lax.div(row_ids, lvec)
            m31 = jnp.full((16,), _D - 1, jnp.int32)
            accs = [jnp.zeros((16,), jnp.float32) for _ in range(4)]
            for d in range(_D):
                dd = (ii + d) & m31
                nv = plsc.load_gather(node_v[k], [row_ids, dd])
                iv = plsc.load_gather(inp_v[k], [b_ids, dd])
                accs[d % 4] = accs[d % 4] + nv * iv
            sc_v[k][pl.ds(q0, 16)] = (accs[0] + accs[1]) + (accs[2] + accs[3])
        b0 = wid * _BW + ci * _CH
        pltpu.sync_copy(sc_v[k], out_hbm.at[pl.ds(b0 * _L, _PPC)])
        cur = nxt


_sc_call = pl.kernel(
    _sc_scores,
    mesh=plsc.VectorSubcoreMesh(core_axis_name="c", subcore_axis_name="s"),
    out_type=jax.ShapeDtypeStruct((_B * _L,), jnp.float32),
    scratch_types=[
        pltpu.VMEM((_CH,), jnp.int32),
        pltpu.VMEM((_CH,), jnp.int32),
        pltpu.VMEM((_PPC,), jnp.int32),
        pltpu.VMEM((_PPC,), jnp.int32),
        pltpu.VMEM((_CH, _D), jnp.float32),
        pltpu.VMEM((_CH, _D), jnp.float32),
        pltpu.VMEM((_PPC, _D), jnp.float32),
        pltpu.VMEM((_PPC, _D), jnp.float32),
        pltpu.VMEM((_PPC,), jnp.float32),
        pltpu.VMEM((_PPC,), jnp.float32),
        pltpu.SemaphoreType.DMA,
        pltpu.SemaphoreType.DMA,
    ],
    compiler_params=pltpu.CompilerParams(
        needs_layout_passes=False, use_tc_tiling_on_sc=False),
)


def _tc_loss(scores_ref, codes_ref, out_ref):
    sc = scores_ref[...]
    cd = codes_ref[...]
    cf = cd.astype(jnp.float32)
    p = 1.0 / (1.0 + jnp.exp(-sc))
    t = cf * jnp.log(p + 1e-7) + (1.0 - cf) * jnp.log(1.0 - p + 1e-7)
    total = jnp.sum(t)
    msum = jnp.sum(jnp.where(cd != -1, 1.0, 0.0))
    out_ref[0, 0] = -total * (msum / float(_B * _L))


def kernel(target_words, context_codes, context_points, in_embed, node_embed):
    tw = target_words.astype(jnp.int32)
    cp = context_points.astype(jnp.int32).reshape(_B * _L)
    tw_p = ((tw & (_WI - 1)) << 2) | (tw >> 18)
    cp_p = ((cp & (_WN - 1)) << 2) | (cp >> 19)
    inT = in_embed.T
    nodeT = node_embed.T
    in_lin = _tr_in(inT, inT, inT, inT).reshape(4 * _WI, _D)
    node_lin = _tr_node(nodeT, nodeT, nodeT, nodeT).reshape(4 * _WN, _D)
    scores = _sc_call(tw_p, cp_p, in_lin, node_lin)
    scores2 = scores.reshape(_B * _L // 128, 128)
    codes2 = context_codes.astype(jnp.int32).reshape(_B * _L // 128, 128)
    out = pl.pallas_call(
        _tc_loss,
        out_shape=jax.ShapeDtypeStruct((1, 1), jnp.float32),
        out_specs=pl.BlockSpec(memory_space=pltpu.SMEM),
    )(scores2, codes2)
    return out[0, 0]


# speedup vs baseline: 4.0473x; 1.4256x over previous
import jax
import jax.numpy as jnp
from jax import lax
from jax.experimental import pallas as pl
from jax.experimental.pallas import tpu as pltpu
from jax.experimental.pallas import tpu_sc as plsc

_VOCAB = 1000000
_D = 32
_B = 16384
_L = 20

_WI = 1 << 18
_WN = 1 << 19
_J = 8192

_NW = 32
_BW = _B // _NW
_CH = 64
_NCH = _BW // _CH
_PPC = _CH * _L
_ROWS = _PPC // 128
_GRP = _PPC // 16


def _tr_body(a_ref, b_ref, c_ref, d_ref, out_ref):
    x = jnp.concatenate(
        [a_ref[...], b_ref[...], c_ref[...], d_ref[...]], axis=0)
    out_ref[...] = x.T


def _make_tr(w, ncols):
    m = w // _J
    last = (ncols + _J - 1) // _J - 1
    return pl.pallas_call(
        _tr_body,
        grid=(m,),
        in_specs=[
            pl.BlockSpec((32, _J), lambda i: (0, i)),
            pl.BlockSpec((32, _J), lambda i: (0, i + m)),
            pl.BlockSpec((32, _J), lambda i: (0, i + 2 * m)),
            pl.BlockSpec((32, _J), lambda i: (0, jnp.minimum(i + 3 * m, last))),
        ],
        out_specs=pl.BlockSpec((_J, 128), lambda i: (i, 0)),
        out_shape=jax.ShapeDtypeStruct((w, 128), jnp.float32),
    )


_tr_in = _make_tr(_WI, _VOCAB)
_tr_node = _make_tr(_WN, 2 * _VOCAB - 1)


def _sc_scores(tw_hbm, cp_hbm, inp_hbm, node_hbm, out_hbm,
               tw_v0, tw_v1, cp_v0, cp_v1, inp_v0, inp_v1,
               node_v0, node_v1, sc_v0, sc_v1, sem0, sem1):
    c = lax.axis_index("c")
    s = lax.axis_index("s")
    wid = s * 2 + c
    tw_v = (tw_v0, tw_v1)
    cp_v = (cp_v0, cp_v1)
    inp_v = (inp_v0, inp_v1)
    node_v = (node_v0, node_v1)
    sc_v = (sc_v0, sc_v1)
    sems = (sem0, sem1)

    def start(ci):
        k = ci % 2
        b0 = wid * _BW + ci * _CH
        pltpu.sync_copy(tw_hbm.at[pl.ds(b0, _CH)], tw_v[k])
        pltpu.sync_copy(cp_hbm.at[pl.ds(b0 * _L, _PPC)], cp_v[k])
        ds = [pltpu.async_copy(inp_hbm.at[tw_v[k]], inp_v[k], sems[k])]
        for j in range(_ROWS):
            ds.append(pltpu.async_copy(
                node_hbm.at[cp_v[k].at[pl.ds(j * 128, 128)]],
                node_v[k].at[pl.ds(j * 128, 128)], sems[k]))
        return ds

    lvec = jnp.full((16,), _L, jnp.int32)
    cur = start(0)
    for ci in range(_NCH):
        k = ci % 2
        nxt = start(ci + 1) if ci + 1 < _NCH else None
        for dd in cur:
            dd.wait()

        @plsc.parallel_loop(0, _GRP, unroll=2)
        def group(g, k=k):
            q0 = g * 16
            ii = lax.iota(jnp.int32, 16)
            row_ids = q0 + ii
            b_ids = lax.div(row_ids, lvec)
            m31 = jnp.full((16,), _D - 1, jnp.int32)
            accs = [jnp.zeros((16,), jnp.float32) for _ in range(4)]
            for d in range(_D):
                dd = (ii + d) & m31
                nv = plsc.load_gather(node_v[k], [row_ids, dd])
                iv = plsc.load_gather(inp_v[k], [b_ids, dd])
                accs[d % 4] = accs[d % 4] + nv * iv
            sc_v[k][pl.ds(q0, 16)] = (accs[0] + accs[1]) + (accs[2] + accs[3])
        b0 = wid * _BW + ci * _CH
        pltpu.sync_copy(sc_v[k], out_hbm.at[pl.ds(b0 * _L, _PPC)])
        cur = nxt


_sc_call = pl.kernel(
    _sc_scores,
    mesh=plsc.VectorSubcoreMesh(core_axis_name="c", subcore_axis_name="s"),
    out_type=jax.ShapeDtypeStruct((_B * _L,), jnp.float32),
    scratch_types=[
        pltpu.VMEM((_CH,), jnp.int32),
        pltpu.VMEM((_CH,), jnp.int32),
        pltpu.VMEM((_PPC,), jnp.int32),
        pltpu.VMEM((_PPC,), jnp.int32),
        pltpu.VMEM((_CH, _D), jnp.float32),
        pltpu.VMEM((_CH, _D), jnp.float32),
        pltpu.VMEM((_PPC, _D), jnp.float32),
        pltpu.VMEM((_PPC, _D), jnp.float32),
        pltpu.VMEM((_PPC,), jnp.float32),
        pltpu.VMEM((_PPC,), jnp.float32),
        pltpu.SemaphoreType.DMA,
        pltpu.SemaphoreType.DMA,
    ],
    compiler_params=pltpu.CompilerParams(
        needs_layout_passes=False, use_tc_tiling_on_sc=False),
)


def _tc_loss(scores_ref, codes_ref, out_ref):
    sc = scores_ref[...]
    cd = codes_ref[...]
    cf = cd.astype(jnp.float32)
    p = 1.0 / (1.0 + jnp.exp(-sc))
    t = cf * jnp.log(p + 1e-7) + (1.0 - cf) * jnp.log(1.0 - p + 1e-7)
    total = jnp.sum(t)
    msum = jnp.sum(jnp.where(cd != -1, 1.0, 0.0))
    out_ref[0, 0] = -total * (msum / float(_B * _L))


def kernel(target_words, context_codes, context_points, in_embed, node_embed):
    tw = target_words.astype(jnp.int32)
    cp = context_points.astype(jnp.int32).reshape(_B * _L)
    tw_p = ((tw & (_WI - 1)) << 2) | (tw >> 18)
    cp_p = ((cp & (_WN - 1)) << 2) | (cp >> 19)
    inT = in_embed.T
    nodeT = node_embed.T
    in_lin = _tr_in(inT, inT, inT, inT).reshape(4 * _WI, _D)
    node_lin = _tr_node(nodeT, nodeT, nodeT, nodeT).reshape(4 * _WN, _D)
    scores = _sc_call(tw_p, cp_p, in_lin, node_lin)
    scores2 = scores.reshape(_B * _L // 128, 128)
    codes2 = context_codes.astype(jnp.int32).reshape(_B * _L // 128, 128)
    out = pl.pallas_call(
        _tc_loss,
        out_shape=jax.ShapeDtypeStruct((1, 1), jnp.float32),
        out_specs=pl.BlockSpec(memory_space=pltpu.SMEM),
    )(scores2, codes2)
    return out[0, 0]


# speedup vs baseline: 4.1472x; 1.0247x over previous
import jax
import jax.numpy as jnp
from jax import lax
from jax.experimental import pallas as pl
from jax.experimental.pallas import tpu as pltpu
from jax.experimental.pallas import tpu_sc as plsc

_VOCAB = 1000000
_D = 32
_B = 16384
_L = 20

_WI = 1 << 18
_WN = 1 << 19
_J = 16384

_NW = 32
_BW = _B // _NW
_CH = 64
_NCH = _BW // _CH
_PPC = _CH * _L
_ROWS = _PPC // 128
_GRP = _PPC // 16


def _tr_body(a_ref, b_ref, c_ref, d_ref, out_ref):
    x = jnp.concatenate(
        [a_ref[...], b_ref[...], c_ref[...], d_ref[...]], axis=0)
    out_ref[...] = x.T


def _make_tr(w, ncols):
    m = w // _J
    last = (ncols + _J - 1) // _J - 1
    return pl.pallas_call(
        _tr_body,
        grid=(m,),
        in_specs=[
            pl.BlockSpec((32, _J), lambda i: (0, i)),
            pl.BlockSpec((32, _J), lambda i: (0, i + m)),
            pl.BlockSpec((32, _J), lambda i: (0, i + 2 * m)),
            pl.BlockSpec((32, _J), lambda i: (0, jnp.minimum(i + 3 * m, last))),
        ],
        out_specs=pl.BlockSpec((_J, 128), lambda i: (i, 0)),
        out_shape=jax.ShapeDtypeStruct((w, 128), jnp.float32),
    )


_tr_in = _make_tr(_WI, _VOCAB)
_tr_node = _make_tr(_WN, 2 * _VOCAB - 1)


def _sc_scores(tw_hbm, cp_hbm, inp_hbm, node_hbm, out_hbm,
               tw_v0, tw_v1, cp_v0, cp_v1, inp_v0, inp_v1,
               node_v0, node_v1, sc_v0, sc_v1, sem0, sem1):
    c = lax.axis_index("c")
    s = lax.axis_index("s")
    wid = s * 2 + c
    tw_v = (tw_v0, tw_v1)
    cp_v = (cp_v0, cp_v1)
    inp_v = (inp_v0, inp_v1)
    node_v = (node_v0, node_v1)
    sc_v = (sc_v0, sc_v1)
    sems = (sem0, sem1)

    def start(ci):
        k = ci % 2
        b0 = wid * _BW + ci * _CH
        pltpu.sync_copy(tw_hbm.at[pl.ds(b0, _CH)], tw_v[k])
        pltpu.sync_copy(cp_hbm.at[pl.ds(b0 * _L, _PPC)], cp_v[k])
        ds = [pltpu.async_copy(inp_hbm.at[tw_v[k]], inp_v[k], sems[k])]
        for j in range(_ROWS):
            ds.append(pltpu.async_copy(
                node_hbm.at[cp_v[k].at[pl.ds(j * 128, 128)]],
                node_v[k].at[pl.ds(j * 128, 128)], sems[k]))
        return ds

    lvec = jnp.full((16,), _L, jnp.int32)
    cur = start(0)
    for ci in range(_NCH):
        k = ci % 2
        nxt = start(ci + 1) if ci + 1 < _NCH else None
        for dd in cur:
            dd.wait()

        @plsc.parallel_loop(0, _GRP, unroll=2)
        def group(g, k=k):
            q0 = g * 16
            ii = lax.iota(jnp.int32, 16)
            row_ids = q0 + ii
            b_ids = lax.div(row_ids, lvec)
            m31 = jnp.full((16,), _D - 1, jnp.int32)
            accs = [jnp.zeros((16,), jnp.float32) for _ in range(4)]
            for d in range(_D):
                dd = (ii + d) & m31
                nv = plsc.load_gather(node_v[k], [row_ids, dd])
                iv = plsc.load_gather(inp_v[k], [b_ids, dd])
                accs[d % 4] = accs[d % 4] + nv * iv
            sc_v[k][pl.ds(q0, 16)] = (accs[0] + accs[1]) + (accs[2] + accs[3])
        b0 = wid * _BW + ci * _CH
        pltpu.sync_copy(sc_v[k], out_hbm.at[pl.ds(b0 * _L, _PPC)])
        cur = nxt


_sc_call = pl.kernel(
    _sc_scores,
    mesh=plsc.VectorSubcoreMesh(core_axis_name="c", subcore_axis_name="s"),
    out_type=jax.ShapeDtypeStruct((_B * _L,), jnp.float32),
    scratch_types=[
        pltpu.VMEM((_CH,), jnp.int32),
        pltpu.VMEM((_CH,), jnp.int32),
        pltpu.VMEM((_PPC,), jnp.int32),
        pltpu.VMEM((_PPC,), jnp.int32),
        pltpu.VMEM((_CH, _D), jnp.float32),
        pltpu.VMEM((_CH, _D), jnp.float32),
        pltpu.VMEM((_PPC, _D), jnp.float32),
        pltpu.VMEM((_PPC, _D), jnp.float32),
        pltpu.VMEM((_PPC,), jnp.float32),
        pltpu.VMEM((_PPC,), jnp.float32),
        pltpu.SemaphoreType.DMA,
        pltpu.SemaphoreType.DMA,
    ],
    compiler_params=pltpu.CompilerParams(
        needs_layout_passes=False, use_tc_tiling_on_sc=False),
)


def _tc_loss(scores_ref, codes_ref, out_ref):
    sc = scores_ref[...]
    cd = codes_ref[...]
    cf = cd.astype(jnp.float32)
    p = 1.0 / (1.0 + jnp.exp(-sc))
    t = cf * jnp.log(p + 1e-7) + (1.0 - cf) * jnp.log(1.0 - p + 1e-7)
    total = jnp.sum(t)
    msum = jnp.sum(jnp.where(cd != -1, 1.0, 0.0))
    out_ref[0, 0] = -total * (msum / float(_B * _L))


def kernel(target_words, context_codes, context_points, in_embed, node_embed):
    tw = target_words.astype(jnp.int32)
    cp = context_points.astype(jnp.int32).reshape(_B * _L)
    tw_p = ((tw & (_WI - 1)) << 2) | (tw >> 18)
    cp_p = ((cp & (_WN - 1)) << 2) | (cp >> 19)
    inT = in_embed.T
    nodeT = node_embed.T
    in_lin = _tr_in(inT, inT, inT, inT).reshape(4 * _WI, _D)
    node_lin = _tr_node(nodeT, nodeT, nodeT, nodeT).reshape(4 * _WN, _D)
    scores = _sc_call(tw_p, cp_p, in_lin, node_lin)
    scores2 = scores.reshape(_B * _L // 128, 128)
    codes2 = context_codes.astype(jnp.int32).reshape(_B * _L // 128, 128)
    out = pl.pallas_call(
        _tc_loss,
        out_shape=jax.ShapeDtypeStruct((1, 1), jnp.float32),
        out_specs=pl.BlockSpec(memory_space=pltpu.SMEM),
    )(scores2, codes2)
    return out[0, 0]


# speedup vs baseline: 4.1882x; 1.0099x over previous
import jax
import jax.numpy as jnp
from jax import lax
from jax.experimental import pallas as pl
from jax.experimental.pallas import tpu as pltpu
from jax.experimental.pallas import tpu_sc as plsc

_VOCAB = 1000000
_D = 32
_B = 16384
_L = 20

_WI = 1 << 18
_WN = 1 << 19
_J = 16384

_NW = 32
_BW = _B // _NW
_CH = 64
_NCH = _BW // _CH
_PPC = _CH * _L
_ROWS = _PPC // 128
_GRP = _PPC // 16


def _tr_body(a_ref, b_ref, c_ref, d_ref, out_ref):
    x = jnp.concatenate(
        [a_ref[...], b_ref[...], c_ref[...], d_ref[...]], axis=0)
    out_ref[...] = x.T


def _make_tr(w, ncols):
    m = w // _J
    last = (ncols + _J - 1) // _J - 1
    return pl.pallas_call(
        _tr_body,
        grid=(m,),
        in_specs=[
            pl.BlockSpec((32, _J), lambda i: (0, i)),
            pl.BlockSpec((32, _J), lambda i: (0, i + m)),
            pl.BlockSpec((32, _J), lambda i: (0, i + 2 * m)),
            pl.BlockSpec((32, _J), lambda i: (0, jnp.minimum(i + 3 * m, last))),
        ],
        out_specs=pl.BlockSpec((_J, 128), lambda i: (i, 0)),
        out_shape=jax.ShapeDtypeStruct((w, 128), jnp.float32),
    )


_tr_in = _make_tr(_WI, _VOCAB)
_tr_node = _make_tr(_WN, 2 * _VOCAB - 1)


def _sc_scores(tw_hbm, cp_hbm, inp_hbm, node_hbm, out_hbm,
               tw_v0, tw_v1, cp_v0, cp_v1, inp_v0, inp_v1,
               node_v0, node_v1, sc_v0, sc_v1, sem0, sem1):
    c = lax.axis_index("c")
    s = lax.axis_index("s")
    wid = s * 2 + c
    tw_v = (tw_v0, tw_v1)
    cp_v = (cp_v0, cp_v1)
    inp_v = (inp_v0, inp_v1)
    node_v = (node_v0, node_v1)
    sc_v = (sc_v0, sc_v1)
    sems = (sem0, sem1)

    def start(ci):
        k = ci % 2
        b0 = wid * _BW + ci * _CH
        pltpu.sync_copy(tw_hbm.at[pl.ds(b0, _CH)], tw_v[k])
        pltpu.sync_copy(cp_hbm.at[pl.ds(b0 * _L, _PPC)], cp_v[k])
        ds = [pltpu.async_copy(inp_hbm.at[tw_v[k]], inp_v[k], sems[k])]
        for j in range(_ROWS):
            ds.append(pltpu.async_copy(
                node_hbm.at[cp_v[k].at[pl.ds(j * 128, 128)]],
                node_v[k].at[pl.ds(j * 128, 128)], sems[k]))
        return ds

    lvec = jnp.full((16,), _L, jnp.int32)
    cur = start(0)
    for ci in range(_NCH):
        k = ci % 2
        nxt = start(ci + 1) if ci + 1 < _NCH else None
        for dd in cur:
            dd.wait()

        @plsc.parallel_loop(0, _GRP, unroll=2)
        def group(g, k=k):
            q0 = g * 16
            ii = lax.iota(jnp.int32, 16)
            row_ids = q0 + ii
            b_ids = lax.div(row_ids, lvec)
            accs = [jnp.zeros((16,), jnp.float32) for _ in range(4)]
            for d in range(_D):
                dd = ii ^ d
                nv = plsc.load_gather(node_v[k], [row_ids, dd])
                iv = plsc.load_gather(inp_v[k], [b_ids, dd])
                accs[d % 4] = accs[d % 4] + nv * iv
            sc_v[k][pl.ds(q0, 16)] = (accs[0] + accs[1]) + (accs[2] + accs[3])
        b0 = wid * _BW + ci * _CH
        pltpu.sync_copy(sc_v[k], out_hbm.at[pl.ds(b0 * _L, _PPC)])
        cur = nxt


_sc_call = pl.kernel(
    _sc_scores,
    mesh=plsc.VectorSubcoreMesh(core_axis_name="c", subcore_axis_name="s"),
    out_type=jax.ShapeDtypeStruct((_B * _L,), jnp.float32),
    scratch_types=[
        pltpu.VMEM((_CH,), jnp.int32),
        pltpu.VMEM((_CH,), jnp.int32),
        pltpu.VMEM((_PPC,), jnp.int32),
        pltpu.VMEM((_PPC,), jnp.int32),
        pltpu.VMEM((_CH, _D), jnp.float32),
        pltpu.VMEM((_CH, _D), jnp.float32),
        pltpu.VMEM((_PPC, _D), jnp.float32),
        pltpu.VMEM((_PPC, _D), jnp.float32),
        pltpu.VMEM((_PPC,), jnp.float32),
        pltpu.VMEM((_PPC,), jnp.float32),
        pltpu.SemaphoreType.DMA,
        pltpu.SemaphoreType.DMA,
    ],
    compiler_params=pltpu.CompilerParams(
        needs_layout_passes=False, use_tc_tiling_on_sc=False),
)


def _tc_loss(scores_ref, codes_ref, out_ref):
    sc = scores_ref[...]
    cd = codes_ref[...]
    cf = cd.astype(jnp.float32)
    p = 1.0 / (1.0 + jnp.exp(-sc))
    t = cf * jnp.log(p + 1e-7) + (1.0 - cf) * jnp.log(1.0 - p + 1e-7)
    total = jnp.sum(t)
    msum = jnp.sum(jnp.where(cd != -1, 1.0, 0.0))
    out_ref[0, 0] = -total * (msum / float(_B * _L))


def kernel(target_words, context_codes, context_points, in_embed, node_embed):
    tw = target_words.astype(jnp.int32)
    cp = context_points.astype(jnp.int32).reshape(_B * _L)
    tw_p = ((tw & (_WI - 1)) << 2) | (tw >> 18)
    cp_p = ((cp & (_WN - 1)) << 2) | (cp >> 19)
    inT = in_embed.T
    nodeT = node_embed.T
    in_lin = _tr_in(inT, inT, inT, inT).reshape(4 * _WI, _D)
    node_lin = _tr_node(nodeT, nodeT, nodeT, nodeT).reshape(4 * _WN, _D)
    scores = _sc_call(tw_p, cp_p, in_lin, node_lin)
    scores2 = scores.reshape(_B * _L // 128, 128)
    codes2 = context_codes.astype(jnp.int32).reshape(_B * _L // 128, 128)
    out = pl.pallas_call(
        _tc_loss,
        out_shape=jax.ShapeDtypeStruct((1, 1), jnp.float32),
        out_specs=pl.BlockSpec(memory_space=pltpu.SMEM),
    )(scores2, codes2)
    return out[0, 0]
